# baseline (speedup 1.0000x reference)
import jax
import jax.numpy as jnp
from jax import lax
from jax.experimental import pallas as pl
from jax.experimental.pallas import tpu as pltpu
from jax.experimental.pallas import tpu_sc as plsc

B = 1024
V = 100000
NCAND = 8192
NNEG = 200
NC, NS = 2, 16
NW = NC * NS
RPW = B // NW
EPW = RPW * NNEG
CHUNK = 128
NCH = EPW // CHUNK


def _sc_body(scores_hbm, obj_hbm, cand_hbm, idx1_hbm,
             a_out, s_out,
             idx_v, neg_v, scores_v, obj_v, gtidx_v, gt_v,
             a_v, s_v, sem, sem2):
    w = lax.axis_index("s") * NC + lax.axis_index("c")
    rbase = w * RPW
    ebase = w * EPW

    pltpu.sync_copy(idx1_hbm.at[pl.ds(ebase, EPW)], idx_v)
    pltpu.sync_copy(obj_hbm.at[pl.ds(rbase, RPW)], obj_v)

    def fire_cand(j, c):
        sl = pl.ds(j * CHUNK, CHUNK)
        pltpu.async_copy(cand_hbm.at[idx_v.at[sl]], neg_v.at[sl], sem)
        return c
    lax.fori_loop(0, NCH, fire_cand, 0)
    pltpu.make_async_copy(cand_hbm.at[pl.ds(0, EPW)], neg_v, sem).wait()

    iota16 = lax.iota(jnp.int32, 16)
    off_lo = iota16 + rbase
    off_hi = off_lo + 16

    def add_off_fire(j, c):
        for k in range(CHUNK // 16):
            sl = pl.ds(j * CHUNK + k * 16, 16)
            idx_v[sl] = (neg_v[sl] << 10) + (off_hi if k % 2 else off_lo)
        sl = pl.ds(j * CHUNK, CHUNK)
        pltpu.async_copy(scores_hbm.at[idx_v.at[sl]], scores_v.at[sl], sem)
        return c

    for g in range(RPW // 16):
        sl = pl.ds(g * 16, 16)
        gtidx_v[sl] = (obj_v[sl] << 10) + (iota16 + (rbase + g * 16))
    pltpu.async_copy(scores_hbm.at[gtidx_v], gt_v, sem2)

    lax.fori_loop(0, NCH, add_off_fire, 0)
    pltpu.make_async_copy(scores_hbm.at[pl.ds(0, EPW)], scores_v, sem).wait()
    pltpu.make_async_copy(scores_hbm.at[pl.ds(0, RPW)], gt_v, sem2).wait()

    def max_body(k, carry):
        m0, m1 = carry
        x0 = scores_v[pl.ds(k * RPW, 16)]
        x1 = scores_v[pl.ds(k * RPW + 16, 16)]
        return (jnp.maximum(m0, x0), jnp.maximum(m1, x1))
    ninf = jnp.full((16,), -jnp.inf, jnp.float32)
    m0, m1 = lax.fori_loop(0, NNEG, max_body, (ninf, ninf))

    def sum_body(k, carry):
        s0, s1 = carry
        x0 = scores_v[pl.ds(k * RPW, 16)]
        x1 = scores_v[pl.ds(k * RPW + 16, 16)]
        return (s0 + jnp.exp(x0 - m0), s1 + jnp.exp(x1 - m1))
    zero = jnp.zeros((16,), jnp.float32)
    s0, s1 = lax.fori_loop(0, NNEG, sum_body, (zero, zero))

    a_v[pl.ds(0, 16)] = m0 - gt_v[pl.ds(0, 16)]
    a_v[pl.ds(16, 16)] = m1 - gt_v[pl.ds(16, 16)]
    s_v[pl.ds(0, 16)] = s0
    s_v[pl.ds(16, 16)] = s1

    pltpu.sync_copy(a_v, a_out.at[pl.ds(rbase, RPW)])
    pltpu.sync_copy(s_v, s_out.at[pl.ds(rbase, RPW)])


_sc_kernel = pl.kernel(
    _sc_body,
    out_type=(jax.ShapeDtypeStruct((B,), jnp.float32),
              jax.ShapeDtypeStruct((B,), jnp.float32)),
    mesh=plsc.VectorSubcoreMesh(core_axis_name="c", subcore_axis_name="s"),
    scratch_types=[
        pltpu.VMEM((EPW,), jnp.int32),
        pltpu.VMEM((EPW,), jnp.int32),
        pltpu.VMEM((EPW,), jnp.float32),
        pltpu.VMEM((RPW,), jnp.int32),
        pltpu.VMEM((RPW,), jnp.int32),
        pltpu.VMEM((RPW,), jnp.float32),
        pltpu.VMEM((RPW,), jnp.float32),
        pltpu.VMEM((RPW,), jnp.float32),
        pltpu.SemaphoreType.DMA,
        pltpu.SemaphoreType.DMA,
    ],
)


def _finish_body(a_ref, s_ref, o_ref):
    o_ref[0, 0] = jnp.mean(a_ref[...] + jnp.log(s_ref[...]))


_finish = pl.pallas_call(
    _finish_body,
    out_shape=jax.ShapeDtypeStruct((1, 1), jnp.float32),
    out_specs=pl.BlockSpec(memory_space=pltpu.SMEM),
)


@jax.jit
def kernel(object_scores, objects, candidates):
    with jax.ensure_compile_time_eval():
        nsi = jax.random.randint(jax.random.key(42), (B * NNEG,), 0, NCAND)
        rows = jnp.arange(B * NNEG, dtype=jnp.int32) // NNEG

        def to_kmajor(a):
            return a.reshape(NW, RPW, NNEG).transpose(0, 2, 1).reshape(-1)
        cand_idx = to_kmajor(nsi.astype(jnp.int32) + rows * NCAND)

    scores_flat = object_scores.T.reshape(-1)
    cand_flat = candidates.astype(jnp.int32).reshape(-1)
    obj = objects.astype(jnp.int32)

    a, s = _sc_kernel(scores_flat, obj, cand_flat, cand_idx)
    out = _finish(a.reshape(8, 128), s.reshape(8, 128))
    return out[0, 0]
